# Initial kernel scaffold; baseline (speedup 1.0000x reference)
import jax
import jax.numpy as jnp
from jax.experimental import pallas as pl


def kernel(xy, W1, b1, W2, b2, Wc, bc):
    raise NotImplementedError("write your pallas kernel here")


# speedup vs baseline: 9.5640x; 9.5640x over previous
import functools

import jax
import jax.numpy as jnp
from jax import lax
from jax.experimental import pallas as pl

K_NB = 20


def _extract_min(u, n):
    m = jnp.min(u, axis=0, keepdims=True)
    tie = u == m
    ii = lax.broadcasted_iota(jnp.int32, (n, n), 0)
    im = jnp.min(jnp.where(tie, ii, n), axis=0, keepdims=True)
    sel = ii == im
    u2 = jnp.where(sel, jnp.inf, u)
    return sel, u2


def _knn_neighbor_max(s, y, k):
    n, c = y.shape
    _, u = _extract_min(s, n)
    acc0 = jnp.full((n, c), -jnp.inf, dtype=jnp.float32)

    def step(_, carry):
        u, acc = carry
        sel, u = _extract_min(u, n)
        g = lax.dot_general(
            sel.astype(jnp.float32), y,
            (((0,), (0,)), ((), ())),
            preferred_element_type=jnp.float32,
        )
        return u, jnp.maximum(acc, g)

    _, acc = lax.fori_loop(0, k, step, (u, acc0))
    return acc


def _leaky(v):
    return jnp.where(v >= 0, v, 0.2 * v)


def _body(xt_ref, w1a_ref, w1d_ref, b1_ref, w2a_ref, w2d_ref, b2_ref,
          wc_ref, bc_ref, out_ref, *, k):
    xt = xt_ref[0]
    n = xt.shape[0]

    y1 = jnp.dot(xt, w1a_ref[...], preferred_element_type=jnp.float32)
    z1 = jnp.dot(xt, w1d_ref[...], preferred_element_type=jnp.float32)
    z1 = z1 + b1_ref[...]
    q1 = jnp.sum(xt * xt, axis=1, keepdims=True)
    inner1 = lax.dot_general(
        xt, xt, (((1,), (1,)), ((), ())), preferred_element_type=jnp.float32)
    s1 = q1 - 2.0 * inner1
    m1 = _knn_neighbor_max(s1, y1, k)
    f1 = _leaky(m1 + z1)

    y2 = jnp.dot(f1, w2a_ref[...], preferred_element_type=jnp.float32)
    z2 = jnp.dot(f1, w2d_ref[...], preferred_element_type=jnp.float32)
    z2 = z2 + b2_ref[...]
    q2 = jnp.sum(f1 * f1, axis=1, keepdims=True)
    inner2 = lax.dot_general(
        f1, f1, (((1,), (1,)), ((), ())), preferred_element_type=jnp.float32)
    s2 = q2 - 2.0 * inner2
    m2 = _knn_neighbor_max(s2, y2, k)
    f2 = _leaky(m2 + z2)

    g1 = jnp.max(f1, axis=0, keepdims=True)
    g2 = jnp.max(f2, axis=0, keepdims=True)
    g = jnp.concatenate([g1, g2], axis=1)
    logits = jnp.dot(g, wc_ref[...], preferred_element_type=jnp.float32)
    logits = logits + bc_ref[...]
    mx = jnp.max(logits, axis=1, keepdims=True)
    sh = logits - mx
    lse = jnp.log(jnp.sum(jnp.exp(sh), axis=1, keepdims=True))
    out_ref[...] = (sh - lse)[None]


def kernel(xy, W1, b1, W2, b2, Wc, bc):
    bs, two, n = xy.shape
    c1 = W1.shape[1]
    c2 = W2.shape[1]
    ncls = Wc.shape[1]

    xt = jnp.transpose(xy, (0, 2, 1))
    W1a, W1d = W1[:two], W1[two:] - W1[:two]
    W2a, W2d = W2[:c1], W2[c1:] - W2[:c1]
    b1r = b1.reshape(1, c1)
    b2r = b2.reshape(1, c2)
    bcr = bc.reshape(1, ncls)

    full = lambda *shape: pl.BlockSpec(shape, lambda b: (0,) * len(shape))
    out = pl.pallas_call(
        functools.partial(_body, k=K_NB),
        grid=(bs,),
        in_specs=[
            pl.BlockSpec((1, n, two), lambda b: (b, 0, 0)),
            full(two, c1), full(two, c1), full(1, c1),
            full(c1, c2), full(c1, c2), full(1, c2),
            full(c1 + c2, ncls), full(1, ncls),
        ],
        out_specs=pl.BlockSpec((1, 1, ncls), lambda b: (b, 0, 0)),
        out_shape=jax.ShapeDtypeStruct((bs, 1, ncls), jnp.float32),
    )(xt, W1a, W1d, b1r, W2a, W2d, b2r, Wc, bcr)
    return out.reshape(bs, ncls)


# speedup vs baseline: 9.9936x; 1.0449x over previous
import functools

import jax
import jax.numpy as jnp
from jax import lax
from jax.experimental import pallas as pl

K_NB = 20


def _extract_min(u, ii, n):
    am = jnp.argmin(u, axis=0)
    sel = ii == am[None, :]
    u2 = jnp.where(sel, jnp.inf, u)
    return sel, u2


def _knn_neighbor_max(s, y, k):
    n, c = y.shape
    ii = lax.broadcasted_iota(jnp.int32, (n, n), 0)
    y_hi = y.astype(jnp.bfloat16)
    y_lo = (y - y_hi.astype(jnp.float32)).astype(jnp.bfloat16)
    _, u = _extract_min(s, ii, n)
    acc0 = jnp.full((n, c), -jnp.inf, dtype=jnp.float32)

    def step(_, carry):
        u, acc = carry
        sel, u = _extract_min(u, ii, n)
        oh = sel.astype(jnp.bfloat16)
        dn = (((0,), (0,)), ((), ()))
        g = lax.dot_general(oh, y_hi, dn, preferred_element_type=jnp.float32)
        g = g + lax.dot_general(oh, y_lo, dn,
                                preferred_element_type=jnp.float32)
        return u, jnp.maximum(acc, g)

    _, acc = lax.fori_loop(0, k, step, (u, acc0))
    return acc


def _leaky(v):
    return jnp.where(v >= 0, v, 0.2 * v)


def _body(xt_ref, w1a_ref, w1d_ref, b1_ref, w2a_ref, w2d_ref, b2_ref,
          wc_ref, bc_ref, out_ref, *, k):
    xt = xt_ref[0]
    n = xt.shape[0]

    y1 = jnp.dot(xt, w1a_ref[...], preferred_element_type=jnp.float32)
    z1 = jnp.dot(xt, w1d_ref[...], preferred_element_type=jnp.float32)
    z1 = z1 + b1_ref[...]
    q1 = jnp.sum(xt * xt, axis=1, keepdims=True)
    inner1 = lax.dot_general(
        xt, xt, (((1,), (1,)), ((), ())), preferred_element_type=jnp.float32)
    s1 = q1 - 2.0 * inner1
    m1 = _knn_neighbor_max(s1, y1, k)
    f1 = _leaky(m1 + z1)

    y2 = jnp.dot(f1, w2a_ref[...], preferred_element_type=jnp.float32)
    z2 = jnp.dot(f1, w2d_ref[...], preferred_element_type=jnp.float32)
    z2 = z2 + b2_ref[...]
    q2 = jnp.sum(f1 * f1, axis=1, keepdims=True)
    inner2 = lax.dot_general(
        f1, f1, (((1,), (1,)), ((), ())), preferred_element_type=jnp.float32)
    s2 = q2 - 2.0 * inner2
    m2 = _knn_neighbor_max(s2, y2, k)
    f2 = _leaky(m2 + z2)

    g1 = jnp.max(f1, axis=0, keepdims=True)
    g2 = jnp.max(f2, axis=0, keepdims=True)
    g = jnp.concatenate([g1, g2], axis=1)
    logits = jnp.dot(g, wc_ref[...], preferred_element_type=jnp.float32)
    logits = logits + bc_ref[...]
    mx = jnp.max(logits, axis=1, keepdims=True)
    sh = logits - mx
    lse = jnp.log(jnp.sum(jnp.exp(sh), axis=1, keepdims=True))
    out_ref[...] = (sh - lse)[None]


def kernel(xy, W1, b1, W2, b2, Wc, bc):
    bs, two, n = xy.shape
    c1 = W1.shape[1]
    c2 = W2.shape[1]
    ncls = Wc.shape[1]

    xt = jnp.transpose(xy, (0, 2, 1))
    W1a, W1d = W1[:two], W1[two:] - W1[:two]
    W2a, W2d = W2[:c1], W2[c1:] - W2[:c1]
    b1r = b1.reshape(1, c1)
    b2r = b2.reshape(1, c2)
    bcr = bc.reshape(1, ncls)

    full = lambda *shape: pl.BlockSpec(shape, lambda b: (0,) * len(shape))
    out = pl.pallas_call(
        functools.partial(_body, k=K_NB),
        grid=(bs,),
        in_specs=[
            pl.BlockSpec((1, n, two), lambda b: (b, 0, 0)),
            full(two, c1), full(two, c1), full(1, c1),
            full(c1, c2), full(c1, c2), full(1, c2),
            full(c1 + c2, ncls), full(1, ncls),
        ],
        out_specs=pl.BlockSpec((1, 1, ncls), lambda b: (b, 0, 0)),
        out_shape=jax.ShapeDtypeStruct((bs, 1, ncls), jnp.float32),
    )(xt, W1a, W1d, b1r, W2a, W2d, b2r, Wc, bcr)
    return out.reshape(bs, ncls)


# speedup vs baseline: 11.5342x; 1.1542x over previous
import functools

import jax
import jax.numpy as jnp
from jax import lax
from jax.experimental import pallas as pl

K_NB = 20


def _extract_min(u, ii, n):
    am = jnp.argmin(u, axis=0)
    sel = ii == am[None, :]
    u2 = jnp.where(sel, jnp.inf, u)
    return sel, u2


def _knn_neighbor_max(s, y, k):
    n, c = y.shape
    ii = lax.broadcasted_iota(jnp.int32, (n, n), 0)
    y_hi = y.astype(jnp.bfloat16)
    y_lo = (y - y_hi.astype(jnp.float32)).astype(jnp.bfloat16)
    y_cat = jnp.concatenate([y_hi, y_lo], axis=1)
    _, u = _extract_min(s, ii, n)
    acc0 = jnp.full((n, c), -jnp.inf, dtype=jnp.float32)

    def step(_, carry):
        u, acc = carry
        sel, u = _extract_min(u, ii, n)
        oh = sel.astype(jnp.bfloat16)
        dn = (((0,), (0,)), ((), ()))
        g2 = lax.dot_general(oh, y_cat, dn,
                             preferred_element_type=jnp.float32)
        g = g2[:, :c] + g2[:, c:]
        return u, jnp.maximum(acc, g)

    _, acc = lax.fori_loop(0, k, step, (u, acc0))
    return acc


def _leaky(v):
    return jnp.where(v >= 0, v, 0.2 * v)


def _body(xt_ref, w1a_ref, w1d_ref, b1_ref, w2a_ref, w2d_ref, b2_ref,
          wc_ref, bc_ref, out_ref, *, k):
    xt = xt_ref[0]
    n = xt.shape[0]

    y1 = jnp.dot(xt, w1a_ref[...], preferred_element_type=jnp.float32)
    z1 = jnp.dot(xt, w1d_ref[...], preferred_element_type=jnp.float32)
    z1 = z1 + b1_ref[...]
    q1 = jnp.sum(xt * xt, axis=1, keepdims=True)
    inner1 = lax.dot_general(
        xt, xt, (((1,), (1,)), ((), ())), preferred_element_type=jnp.float32)
    s1 = q1 - 2.0 * inner1
    m1 = _knn_neighbor_max(s1, y1, k)
    f1 = _leaky(m1 + z1)

    y2 = jnp.dot(f1, w2a_ref[...], preferred_element_type=jnp.float32)
    z2 = jnp.dot(f1, w2d_ref[...], preferred_element_type=jnp.float32)
    z2 = z2 + b2_ref[...]
    q2 = jnp.sum(f1 * f1, axis=1, keepdims=True)
    inner2 = lax.dot_general(
        f1, f1, (((1,), (1,)), ((), ())), preferred_element_type=jnp.float32)
    s2 = q2 - 2.0 * inner2
    m2 = _knn_neighbor_max(s2, y2, k)
    f2 = _leaky(m2 + z2)

    g1 = jnp.max(f1, axis=0, keepdims=True)
    g2 = jnp.max(f2, axis=0, keepdims=True)
    g = jnp.concatenate([g1, g2], axis=1)
    logits = jnp.dot(g, wc_ref[...], preferred_element_type=jnp.float32)
    logits = logits + bc_ref[...]
    mx = jnp.max(logits, axis=1, keepdims=True)
    sh = logits - mx
    lse = jnp.log(jnp.sum(jnp.exp(sh), axis=1, keepdims=True))
    out_ref[...] = (sh - lse)[None]


def kernel(xy, W1, b1, W2, b2, Wc, bc):
    bs, two, n = xy.shape
    c1 = W1.shape[1]
    c2 = W2.shape[1]
    ncls = Wc.shape[1]

    xt = jnp.transpose(xy, (0, 2, 1))
    W1a, W1d = W1[:two], W1[two:] - W1[:two]
    W2a, W2d = W2[:c1], W2[c1:] - W2[:c1]
    b1r = b1.reshape(1, c1)
    b2r = b2.reshape(1, c2)
    bcr = bc.reshape(1, ncls)

    full = lambda *shape: pl.BlockSpec(shape, lambda b: (0,) * len(shape))
    out = pl.pallas_call(
        functools.partial(_body, k=K_NB),
        grid=(bs,),
        in_specs=[
            pl.BlockSpec((1, n, two), lambda b: (b, 0, 0)),
            full(two, c1), full(two, c1), full(1, c1),
            full(c1, c2), full(c1, c2), full(1, c2),
            full(c1 + c2, ncls), full(1, ncls),
        ],
        out_specs=pl.BlockSpec((1, 1, ncls), lambda b: (b, 0, 0)),
        out_shape=jax.ShapeDtypeStruct((bs, 1, ncls), jnp.float32),
    )(xt, W1a, W1d, b1r, W2a, W2d, b2r, Wc, bcr)
    return out.reshape(bs, ncls)


# speedup vs baseline: 16.5169x; 1.4320x over previous
import functools

import jax
import jax.numpy as jnp
from jax import lax
from jax.experimental import pallas as pl
from jax.experimental.pallas import tpu as pltpu
from jax.experimental.pallas import tpu_sc as plsc

K_NB = 20
KPAD = 24


def _extract_idx(s, n, k):
    ii = lax.broadcasted_iota(jnp.int32, (n, n), 0)
    am = jnp.argmin(s, axis=0)
    u = jnp.where(ii == am[None, :], jnp.inf, s)
    idx_rows = []
    for _ in range(k):
        am = jnp.argmin(u, axis=0)
        idx_rows.append(am)
        u = jnp.where(ii == am[None, :], jnp.inf, u)
    return jnp.stack(idx_rows, axis=0)


def _leaky(v):
    return jnp.where(v >= 0, v, 0.2 * v)


def _stage_a(xt_ref, w1a_ref, y_ref, idx_ref, *, k):
    b = pl.program_id(0)
    xt = xt_ref[0]
    n = xt.shape[0]
    y1 = jnp.dot(xt, w1a_ref[...], preferred_element_type=jnp.float32)
    tw = y_ref.shape[1]
    y_ref[...] = jnp.concatenate(
        [y1, jnp.zeros((n, tw - y1.shape[1]), jnp.float32)], axis=1)
    q = jnp.sum(xt * xt, axis=1, keepdims=True)
    inner = lax.dot_general(
        xt, xt, (((1,), (1,)), ((), ())), preferred_element_type=jnp.float32)
    s = q - 2.0 * inner
    idx = _extract_idx(s, n, k) + b * n
    pad = jnp.zeros((KPAD - k, n), jnp.int32)
    idx_ref[...] = jnp.concatenate([idx, pad], axis=0)[None]


def _stage_b(xt_ref, o1_ref, w1d_ref, b1_ref, w2a_ref,
             f1_ref, y2_ref, idx_ref, *, k):
    b = pl.program_id(0)
    xt = xt_ref[0]
    n = xt.shape[0]
    z1 = jnp.dot(xt, w1d_ref[...], preferred_element_type=jnp.float32)
    c1 = z1.shape[1]
    f1 = _leaky(o1_ref[:, :c1] + z1 + b1_ref[...])
    f1_ref[...] = f1
    y2_ref[...] = jnp.dot(f1, w2a_ref[...], preferred_element_type=jnp.float32)
    q = jnp.sum(f1 * f1, axis=1, keepdims=True)
    inner = lax.dot_general(
        f1, f1, (((1,), (1,)), ((), ())), preferred_element_type=jnp.float32)
    s = q - 2.0 * inner
    idx = _extract_idx(s, n, k) + b * n
    pad = jnp.zeros((KPAD - k, n), jnp.int32)
    idx_ref[...] = jnp.concatenate([idx, pad], axis=0)[None]


def _stage_c(o2_ref, f1_ref, w2d_ref, b2_ref, wc_ref, bc_ref, out_ref):
    f1 = f1_ref[...]
    z2 = jnp.dot(f1, w2d_ref[...], preferred_element_type=jnp.float32)
    f2 = _leaky(o2_ref[...] + z2 + b2_ref[...])
    g1 = jnp.max(f1, axis=0, keepdims=True)
    g2 = jnp.max(f2, axis=0, keepdims=True)
    g = jnp.concatenate([g1, g2], axis=1)
    logits = jnp.dot(g, wc_ref[...], preferred_element_type=jnp.float32)
    logits = logits + bc_ref[...]
    mx = jnp.max(logits, axis=1, keepdims=True)
    sh = logits - mx
    lse = jnp.log(jnp.sum(jnp.exp(sh), axis=1, keepdims=True))
    out_ref[...] = (sh - lse)[None]


def _sc_gather_max(table, idx, c, ch, n):
    P = table.shape[0]
    info = plsc.get_sparse_core_info()
    nw = info.num_cores * info.num_subcores
    per_w = P // nw
    ch = min(ch, per_w)
    n_chunks = per_w // ch
    mesh = plsc.VectorSubcoreMesh(core_axis_name="c", subcore_axis_name="s")

    @functools.partial(
        pl.kernel, mesh=mesh,
        out_type=jax.ShapeDtypeStruct((P, c), jnp.float32),
        scratch_types=[
            pltpu.VMEM((K_NB, ch), jnp.int32),
            pltpu.VMEM((K_NB, ch, c), jnp.float32),
            pltpu.VMEM((ch, c), jnp.float32),
            pltpu.SemaphoreType.DMA,
        ],
    )
    def run(table_hbm, idx_hbm, out_hbm, idx_v, rows_v, acc_v, sem):
        wid = lax.axis_index("s") * info.num_cores + lax.axis_index("c")
        base = wid * per_w

        def chunk(ci, carry):
            pbase = base + ci * ch
            bb = pbase // n
            col = pbase - bb * n
            for kk in range(K_NB):
                off = (bb * KPAD + kk) * n + col
                pltpu.sync_copy(idx_hbm.at[pl.ds(off, ch)], idx_v.at[kk])
            copies = [
                pltpu.async_copy(table_hbm.at[idx_v.at[kk]], rows_v.at[kk],
                                 sem)
                for kk in range(K_NB)
            ]
            for cp in copies:
                cp.wait()

            def point(p, carry2):
                for d in range(c // 16):
                    sl = pl.ds(d * 16, 16)
                    acc = rows_v[0, p, sl]
                    for kk in range(1, K_NB):
                        acc = jnp.maximum(acc, rows_v[kk, p, sl])
                    acc_v[p, sl] = acc
                return carry2

            lax.fori_loop(0, ch, point, 0, unroll=False)
            pltpu.sync_copy(acc_v, out_hbm.at[pl.ds(pbase, ch)])
            return carry

        lax.fori_loop(0, n_chunks, chunk, 0, unroll=False)

    return run(table, idx)


def kernel(xy, W1, b1, W2, b2, Wc, bc):
    bs, two, n = xy.shape
    c1 = W1.shape[1]
    c2 = W2.shape[1]
    ncls = Wc.shape[1]
    P = bs * n

    xt = jnp.transpose(xy, (0, 2, 1))
    W1a, W1d = W1[:two], W1[two:] - W1[:two]
    W2a, W2d = W2[:c1], W2[c1:] - W2[:c1]
    b1r = b1.reshape(1, c1)
    b2r = b2.reshape(1, c2)
    bcr = bc.reshape(1, ncls)

    full = lambda *shape: pl.BlockSpec(shape, lambda b: (0,) * len(shape))
    xt_spec = pl.BlockSpec((1, n, two), lambda b: (b, 0, 0))
    row_spec = lambda c: pl.BlockSpec((n, c), lambda b: (b, 0))
    idx_spec = pl.BlockSpec((1, KPAD, n), lambda b: (b, 0, 0))

    y1t, idx1 = pl.pallas_call(
        functools.partial(_stage_a, k=K_NB),
        grid=(bs,),
        in_specs=[xt_spec, full(two, c1)],
        out_specs=[row_spec(c2), idx_spec],
        out_shape=[
            jax.ShapeDtypeStruct((P, c2), jnp.float32),
            jax.ShapeDtypeStruct((bs, KPAD, n), jnp.int32),
        ],
    )(xt, W1a)

    o1 = _sc_gather_max(y1t, idx1.reshape(-1), c2, 32, n)

    f1t, y2t, idx2 = pl.pallas_call(
        functools.partial(_stage_b, k=K_NB),
        grid=(bs,),
        in_specs=[xt_spec, row_spec(c2), full(two, c1), full(1, c1),
                  full(c1, c2)],
        out_specs=[row_spec(c1), row_spec(c2), idx_spec],
        out_shape=[
            jax.ShapeDtypeStruct((P, c1), jnp.float32),
            jax.ShapeDtypeStruct((P, c2), jnp.float32),
            jax.ShapeDtypeStruct((bs, KPAD, n), jnp.int32),
        ],
    )(xt, o1, W1d, b1r, W2a)

    o2 = _sc_gather_max(y2t, idx2.reshape(-1), c2, 32, n)

    out = pl.pallas_call(
        _stage_c,
        grid=(bs,),
        in_specs=[row_spec(c2), row_spec(c1), full(c1, c2), full(1, c2),
                  full(c1 + c2, ncls), full(1, ncls)],
        out_specs=pl.BlockSpec((1, 1, ncls), lambda b: (b, 0, 0)),
        out_shape=jax.ShapeDtypeStruct((bs, 1, ncls), jnp.float32),
    )(o2, f1t, W2d, b2r, Wc, bcr)
    return out.reshape(bs, ncls)


# speedup vs baseline: 21.5105x; 1.3023x over previous
import functools

import jax
import jax.numpy as jnp
from jax import lax
from jax.experimental import pallas as pl
from jax.experimental.pallas import tpu as pltpu
from jax.experimental.pallas import tpu_sc as plsc

K_NB = 20
KPAD = 24


def _extract_idx(s, n, k):
    ii = lax.broadcasted_iota(jnp.int32, (n, n), 0)
    am = jnp.argmin(s, axis=0)
    u = jnp.where(ii == am[None, :], jnp.inf, s)
    idx_rows = []
    for _ in range(k):
        am = jnp.argmin(u, axis=0)
        idx_rows.append(am)
        u = jnp.where(ii == am[None, :], jnp.inf, u)
    return jnp.stack(idx_rows, axis=0)


def _leaky(v):
    return jnp.where(v >= 0, v, 0.2 * v)


def _stage_a(xt_ref, w1a_ref, y_ref, idx_ref, *, k):
    b = pl.program_id(0)
    xt = xt_ref[0]
    n = xt.shape[0]
    y1 = jnp.dot(xt, w1a_ref[...], preferred_element_type=jnp.float32)
    tw = y_ref.shape[1]
    y_ref[...] = jnp.concatenate(
        [y1, jnp.zeros((n, tw - y1.shape[1]), jnp.float32)], axis=1)
    q = jnp.sum(xt * xt, axis=1, keepdims=True)
    inner = lax.dot_general(
        xt, xt, (((1,), (1,)), ((), ())), preferred_element_type=jnp.float32)
    s = q - 2.0 * inner
    idx = _extract_idx(s, n, k) + b * n
    pad = jnp.zeros((KPAD - k, n), jnp.int32)
    idx_ref[...] = jnp.concatenate([idx, pad], axis=0)[None]


def _stage_b(xt_ref, o1_ref, w1d_ref, b1_ref, w2a_ref,
             f1_ref, y2_ref, idx_ref, *, k):
    b = pl.program_id(0)
    xt = xt_ref[0]
    n = xt.shape[0]
    z1 = jnp.dot(xt, w1d_ref[...], preferred_element_type=jnp.float32)
    c1 = z1.shape[1]
    f1 = _leaky(o1_ref[:, :c1] + z1 + b1_ref[...])
    f1_ref[...] = f1
    y2_ref[...] = jnp.dot(f1, w2a_ref[...], preferred_element_type=jnp.float32)
    q = jnp.sum(f1 * f1, axis=1, keepdims=True)
    inner = lax.dot_general(
        f1, f1, (((1,), (1,)), ((), ())), preferred_element_type=jnp.float32)
    s = q - 2.0 * inner
    idx = _extract_idx(s, n, k) + b * n
    pad = jnp.zeros((KPAD - k, n), jnp.int32)
    idx_ref[...] = jnp.concatenate([idx, pad], axis=0)[None]


def _stage_c(o2_ref, f1_ref, w2d_ref, b2_ref, wc_ref, bc_ref, out_ref):
    f1 = f1_ref[...]
    z2 = jnp.dot(f1, w2d_ref[...], preferred_element_type=jnp.float32)
    f2 = _leaky(o2_ref[...] + z2 + b2_ref[...])
    g1 = jnp.max(f1, axis=0, keepdims=True)
    g2 = jnp.max(f2, axis=0, keepdims=True)
    g = jnp.concatenate([g1, g2], axis=1)
    logits = jnp.dot(g, wc_ref[...], preferred_element_type=jnp.float32)
    logits = logits + bc_ref[...]
    mx = jnp.max(logits, axis=1, keepdims=True)
    sh = logits - mx
    lse = jnp.log(jnp.sum(jnp.exp(sh), axis=1, keepdims=True))
    out_ref[...] = (sh - lse)[None]


def _sc_gather_max(table, idx, c, ch, n):
    P = table.shape[0]
    info = plsc.get_sparse_core_info()
    nw = info.num_cores * info.num_subcores
    per_w = P // nw
    ch = min(ch, per_w)
    n_chunks = per_w // ch
    mesh = plsc.VectorSubcoreMesh(core_axis_name="c", subcore_axis_name="s")

    @functools.partial(
        pl.kernel, mesh=mesh,
        out_type=jax.ShapeDtypeStruct((P, c), jnp.float32),
        scratch_types=[
            pltpu.VMEM((K_NB * n,), jnp.int32),
            pltpu.VMEM((K_NB, ch, c), jnp.float32),
            pltpu.VMEM((ch, c), jnp.float32),
            pltpu.SemaphoreType.DMA,
        ],
    )
    def run(table_hbm, idx_hbm, out_hbm, idx_vb, rows_v, acc_v, sem):
        wid = lax.axis_index("s") * info.num_cores + lax.axis_index("c")
        base = wid * per_w
        nb = per_w // n

        def batch(bi, carry):
            bb = base // n + bi
            for kk in range(K_NB):
                pltpu.sync_copy(idx_hbm.at[pl.ds((bb * KPAD + kk) * n, n)],
                                idx_vb.at[pl.ds(kk * n, n)])

            def chunk(ci, carry2):
                col = ci * ch
                copies = [
                    pltpu.async_copy(
                        table_hbm.at[idx_vb.at[pl.ds(kk * n + col, ch)]],
                        rows_v.at[kk], sem)
                    for kk in range(K_NB)
                ]
                for cp in copies:
                    cp.wait()

                def point(p, carry3):
                    for d in range(c // 16):
                        sl = pl.ds(d * 16, 16)
                        acc = rows_v[0, p, sl]
                        for kk in range(1, K_NB):
                            acc = jnp.maximum(acc, rows_v[kk, p, sl])
                        acc_v[p, sl] = acc
                    return carry3

                lax.fori_loop(0, ch, point, 0, unroll=False)
                pltpu.sync_copy(acc_v, out_hbm.at[pl.ds(bb * n + col, ch)])
                return carry2

            lax.fori_loop(0, n // ch, chunk, 0, unroll=False)
            return carry

        lax.fori_loop(0, nb, batch, 0, unroll=False)

    return run(table, idx)


def kernel(xy, W1, b1, W2, b2, Wc, bc):
    bs, two, n = xy.shape
    c1 = W1.shape[1]
    c2 = W2.shape[1]
    ncls = Wc.shape[1]
    P = bs * n

    xt = jnp.transpose(xy, (0, 2, 1))
    W1a, W1d = W1[:two], W1[two:] - W1[:two]
    W2a, W2d = W2[:c1], W2[c1:] - W2[:c1]
    b1r = b1.reshape(1, c1)
    b2r = b2.reshape(1, c2)
    bcr = bc.reshape(1, ncls)

    full = lambda *shape: pl.BlockSpec(shape, lambda b: (0,) * len(shape))
    xt_spec = pl.BlockSpec((1, n, two), lambda b: (b, 0, 0))
    row_spec = lambda c: pl.BlockSpec((n, c), lambda b: (b, 0))
    idx_spec = pl.BlockSpec((1, KPAD, n), lambda b: (b, 0, 0))

    y1t, idx1 = pl.pallas_call(
        functools.partial(_stage_a, k=K_NB),
        grid=(bs,),
        in_specs=[xt_spec, full(two, c1)],
        out_specs=[row_spec(c2), idx_spec],
        out_shape=[
            jax.ShapeDtypeStruct((P, c2), jnp.float32),
            jax.ShapeDtypeStruct((bs, KPAD, n), jnp.int32),
        ],
    )(xt, W1a)

    o1 = _sc_gather_max(y1t, idx1.reshape(-1), c2, 32, n)

    f1t, y2t, idx2 = pl.pallas_call(
        functools.partial(_stage_b, k=K_NB),
        grid=(bs,),
        in_specs=[xt_spec, row_spec(c2), full(two, c1), full(1, c1),
                  full(c1, c2)],
        out_specs=[row_spec(c1), row_spec(c2), idx_spec],
        out_shape=[
            jax.ShapeDtypeStruct((P, c1), jnp.float32),
            jax.ShapeDtypeStruct((P, c2), jnp.float32),
            jax.ShapeDtypeStruct((bs, KPAD, n), jnp.int32),
        ],
    )(xt, o1, W1d, b1r, W2a)

    o2 = _sc_gather_max(y2t, idx2.reshape(-1), c2, 32, n)

    out = pl.pallas_call(
        _stage_c,
        grid=(bs,),
        in_specs=[row_spec(c2), row_spec(c1), full(c1, c2), full(1, c2),
                  full(c1 + c2, ncls), full(1, ncls)],
        out_specs=pl.BlockSpec((1, 1, ncls), lambda b: (b, 0, 0)),
        out_shape=jax.ShapeDtypeStruct((bs, 1, ncls), jnp.float32),
    )(o2, f1t, W2d, b2r, Wc, bcr)
    return out.reshape(bs, ncls)
